# baseline (speedup 1.0000x reference)
import functools

import jax
import jax.numpy as jnp
from jax import lax
from jax.experimental import pallas as pl
from jax.experimental.pallas import tpu as pltpu
from jax.experimental.pallas import tpu_sc as plsc

_LANE = 128


@functools.lru_cache(maxsize=None)
def _make_gather(seq, n_batch, vocab, dim):
    info = plsc.get_sparse_core_info()
    nc, ns = info.num_cores, info.num_subcores
    nw = nc * ns
    n_blk = n_batch // _LANE
    d_blk = dim // 8
    assert n_blk == nw and seq % 2 == 0

    mesh = plsc.VectorSubcoreMesh(core_axis_name="c", subcore_axis_name="s")

    @functools.partial(
        pl.kernel,
        mesh=mesh,
        out_type=jax.ShapeDtypeStruct((seq, d_blk, n_blk, 8, _LANE), jnp.float32),
        scratch_types=[
            pltpu.VMEM((seq, _LANE), jnp.int32),
            pltpu.VMEM((_LANE, dim), jnp.float32),
            pltpu.VMEM((_LANE, dim), jnp.float32),
            pltpu.VMEM((d_blk, 8, _LANE), jnp.float32),
            pltpu.VMEM((d_blk, 8, _LANE), jnp.float32),
            pltpu.SemaphoreType.DMA,
            pltpu.SemaphoreType.DMA,
            pltpu.SemaphoreType.DMA,
            pltpu.SemaphoreType.DMA,
        ],
        compiler_params=pltpu.CompilerParams(
            use_tc_tiling_on_sc=False, needs_layout_passes=False
        ),
    )
    def gather_kernel(ids_hbm, table_hbm, out_hbm, idx_v, rows0, rows1,
                      tile0, tile1, gs0, gs1, ws0, ws1):
        wid = lax.axis_index("s") * nc + lax.axis_index("c")
        pltpu.sync_copy(ids_hbm.at[:, pl.ds(wid * _LANE, _LANE)], idx_v)

        rows = (rows0, rows1)
        tiles = (tile0, tile1)
        gsems = (gs0, gs1)
        wsems = (ws0, ws1)
        iotas = tuple(
            lax.iota(jnp.int32, 16) + 16 * cb for cb in range(8)
        )

        def gfire(s, b):
            pltpu.async_copy(table_hbm.at[idx_v.at[s]], rows[b], gsems[b])

        def gwait(b):
            pltpu.make_async_copy(
                table_hbm.at[idx_v.at[0]], rows[b], gsems[b]
            ).wait()

        def wfire(s, b):
            pltpu.async_copy(tiles[b], out_hbm.at[s, :, wid], wsems[b])

        def wwait(b):
            pltpu.make_async_copy(
                tiles[b], out_hbm.at[0, :, 0], wsems[b]
            ).wait()

        def transpose(b):
            rv = rows[b]
            tv = tiles[b]

            def dbody(i, carry):
                for r in range(8):
                    d = 8 * i + r
                    col = jnp.zeros((16,), jnp.int32) + d
                    for cb in range(8):
                        vals = plsc.load_gather(rv, [iotas[cb], col])
                        tv[i, r, pl.ds(16 * cb, 16)] = vals
                return carry

            lax.fori_loop(0, d_blk, dbody, 0)

        gfire(0, 0)
        gfire(1, 1)

        for b in range(2):
            gwait(b)
            transpose(b)
            wfire(b, b)
            gfire(b + 2, b)

        def body(s2, carry):
            for b in range(2):
                s = 2 * s2 + b
                gwait(b)
                wwait(b)
                transpose(b)
                wfire(s, b)
                gfire(s + 2, b)
            return carry

        lax.fori_loop(1, seq // 2 - 1, body, 0)

        for b in range(2):
            s = seq - 2 + b
            gwait(b)
            wwait(b)
            transpose(b)
            wfire(s, b)
        wwait(0)
        wwait(1)

    return gather_kernel


def kernel(input_ids, weight):
    n_batch, seq = input_ids.shape
    vocab, dim = weight.shape
    ids_t = input_ids.T
    out5d = _make_gather(seq, n_batch, vocab, dim)(ids_t, weight)
    return out5d.transpose((2, 4, 0, 1, 3)).reshape(n_batch, seq, dim)


# speedup vs baseline: 1.8858x; 1.8858x over previous
import functools

import jax
import jax.numpy as jnp
from jax import lax
from jax.experimental import pallas as pl
from jax.experimental.pallas import tpu as pltpu
from jax.experimental.pallas import tpu_sc as plsc

_LANE = 128


@functools.lru_cache(maxsize=None)
def _make_gather(seq, n_batch, vocab, dim):
    info = plsc.get_sparse_core_info()
    nc, ns = info.num_cores, info.num_subcores
    nw = nc * ns
    n_blk = n_batch // _LANE
    d_blk = dim // 8
    assert n_blk == nw and seq % 2 == 0

    mesh = plsc.VectorSubcoreMesh(core_axis_name="c", subcore_axis_name="s")

    @functools.partial(
        pl.kernel,
        mesh=mesh,
        out_type=jax.ShapeDtypeStruct((seq, d_blk, n_blk, 8, _LANE), jnp.float32),
        scratch_types=[
            pltpu.VMEM((seq, _LANE), jnp.int32),
            pltpu.VMEM((_LANE, dim), jnp.float32),
            pltpu.VMEM((_LANE, dim), jnp.float32),
            pltpu.VMEM((d_blk, 8, _LANE + 1), jnp.float32),
            pltpu.VMEM((d_blk, 8, _LANE + 1), jnp.float32),
            pltpu.SemaphoreType.DMA,
            pltpu.SemaphoreType.DMA,
            pltpu.SemaphoreType.DMA,
            pltpu.SemaphoreType.DMA,
        ],
        compiler_params=pltpu.CompilerParams(
            use_tc_tiling_on_sc=False, needs_layout_passes=False
        ),
    )
    def gather_kernel(ids_hbm, table_hbm, out_hbm, idx_v, rows0, rows1,
                      tile0, tile1, gs0, gs1, ws0, ws1):
        wid = lax.axis_index("s") * nc + lax.axis_index("c")
        pltpu.sync_copy(ids_hbm.at[:, pl.ds(wid * _LANE, _LANE)], idx_v)

        rows = (rows0, rows1)
        tiles = (tile0, tile1)
        gsems = (gs0, gs1)
        wsems = (ws0, ws1)
        iota = lax.iota(jnp.int32, 16)
        dblocks = tuple(
            (iota // 8 + (16 * db) // 8, iota % 8, 16 * db)
            for db in range(dim // 16)
        )

        def gfire(s, b):
            pltpu.async_copy(table_hbm.at[idx_v.at[s]], rows[b], gsems[b])

        def gwait(b):
            pltpu.make_async_copy(
                table_hbm.at[idx_v.at[0]], rows[b], gsems[b]
            ).wait()

        def wfire(s, b):
            pltpu.async_copy(
                tiles[b].at[:, :, pl.ds(0, _LANE)],
                out_hbm.at[s, :, wid],
                wsems[b],
            )

        def wwait(b):
            pltpu.make_async_copy(
                tiles[b].at[:, :, pl.ds(0, _LANE)],
                out_hbm.at[0, :, 0],
                wsems[b],
            ).wait()

        def transpose(b):
            rv = rows[b]
            tv = tiles[b]

            def cbody(c8, carry):
                c0 = 8 * c8
                for dc in range(8):
                    c = c0 + dc
                    cvec = jnp.zeros((16,), jnp.int32) + c
                    for i_idx, r_idx, d0 in dblocks:
                        vals = rv[c, pl.ds(d0, 16)]
                        plsc.store_scatter(tv, [i_idx, r_idx, cvec], vals)
                return carry

            lax.fori_loop(0, _LANE // 8, cbody, 0)

        gfire(0, 0)
        gfire(1, 1)

        for b in range(2):
            gwait(b)
            transpose(b)
            wfire(b, b)
            gfire(b + 2, b)

        def body(s2, carry):
            for b in range(2):
                s = 2 * s2 + b
                gwait(b)
                wwait(b)
                transpose(b)
                wfire(s, b)
                gfire(s + 2, b)
            return carry

        lax.fori_loop(1, seq // 2 - 1, body, 0)

        for b in range(2):
            s = seq - 2 + b
            gwait(b)
            wwait(b)
            transpose(b)
            wfire(s, b)
        wwait(0)
        wwait(1)

    return gather_kernel


def kernel(input_ids, weight):
    n_batch, seq = input_ids.shape
    vocab, dim = weight.shape
    ids_t = input_ids.T
    out5d = _make_gather(seq, n_batch, vocab, dim)(ids_t, weight)
    return out5d.transpose((2, 4, 0, 1, 3)).reshape(n_batch, seq, dim)
